# Initial kernel scaffold; baseline (speedup 1.0000x reference)
import jax
import jax.numpy as jnp
from jax.experimental import pallas as pl


def kernel(feature_batch, memory_bank):
    raise NotImplementedError("write your pallas kernel here")


# speedup vs baseline: 8.0476x; 8.0476x over previous
import functools

import jax
import jax.numpy as jnp
from jax.experimental import pallas as pl
from jax.experimental.pallas import tpu as pltpu

B_IMGS = 2
HW = 1024
C = 128
N_BANK = 50000
G = 16
LANES = 128
TB = G * LANES
T_STEPS = (N_BANK + TB - 1) // TB
N_PAD = T_STEPS * TB
K = 9
BIG = 3.0e38


def _oems_pairs(n):
    pairs = []

    def merge(lo, n2, r):
        step = r * 2
        if step < n2:
            merge(lo, n2, step)
            merge(lo + r, n2, step)
            for i in range(lo + r, lo + n2 - r, step):
                pairs.append((i, i + r))
        else:
            pairs.append((lo, lo + r))

    def sort_range(lo, hi):
        if (hi - lo) >= 1:
            mid = lo + ((hi - lo) // 2)
            sort_range(lo, mid)
            sort_range(mid + 1, hi)
            merge(lo, hi - lo + 1, 1)

    sort_range(0, n - 1)
    return pairs


_SORT_PAIRS = _oems_pairs(G)


def _scorer_body(fv_ref, bankt_ref, pix_ref, img_ref, run_ref):
    t = pl.program_id(1)
    fv = fv_ref[...]
    bankt = bankt_ref[...]

    m2 = jnp.sum(bankt * bankt, axis=0, keepdims=True)
    col = t * TB + jax.lax.broadcasted_iota(jnp.int32, (1, TB), 1)
    m2 = jnp.where(col < N_BANK, m2, BIG)

    qm = jnp.dot(fv * jnp.float32(-2.0), bankt,
                 preferred_element_type=jnp.float32)
    d = qm + m2

    v = [d[:, j * LANES:(j + 1) * LANES] for j in range(G)]
    for (i, j) in _SORT_PAIRS:
        lo = jnp.minimum(v[i], v[j])
        hi = jnp.maximum(v[i], v[j])
        v[i] = lo
        v[j] = hi

    @pl.when(t == 0)
    def _init():
        run_ref[...] = jnp.concatenate(v, axis=1)

    @pl.when(t > 0)
    def _merge():
        r = [run_ref[:, j * LANES:(j + 1) * LANES] for j in range(G)]
        c = [jnp.minimum(r[j], v[G - 1 - j]) for j in range(G)]
        for dist in (8, 4, 2, 1):
            for base in range(0, G, dist * 2):
                for i2 in range(base, base + dist):
                    lo = jnp.minimum(c[i2], c[i2 + dist])
                    hi = jnp.maximum(c[i2], c[i2 + dist])
                    c[i2] = lo
                    c[i2 + dist] = hi
        run_ref[...] = jnp.concatenate(c, axis=1)

    @pl.when(t == T_STEPS - 1)
    def _final():
        x = run_ref[...]
        q2 = jnp.sum(fv * fv, axis=1, keepdims=True)
        iota_l = jax.lax.broadcasted_iota(jnp.int32, (HW, TB), 1)
        big_i = jnp.int32(2 ** 30)

        vals = []
        for _ in range(K):
            m = jnp.min(x, axis=1, keepdims=True)
            pos = jnp.min(jnp.where(x == m, iota_l, big_i), axis=1, keepdims=True)
            x = jnp.where(iota_l == pos, BIG, x)
            vals.append(m)

        s = [jnp.sqrt(jnp.maximum(vv + q2, jnp.float32(0.0))) for vv in vals]

        pix_ref[...] = s[0]

        mx = jnp.max(s[0])
        iota_r = jax.lax.broadcasted_iota(jnp.int32, (HW, 1), 0)
        pos_r = jnp.min(jnp.where(s[0] == mx, iota_r, big_i))
        sel = [jnp.sum(jnp.where(iota_r == pos_r, si, jnp.float32(0.0)))
               for si in s]
        e = [jnp.exp(si - sel[K - 1]) for si in sel]
        denom = e[0]
        for ei in e[1:]:
            denom = denom + ei
        img = sel[0] * (jnp.float32(1.0) - e[0] / denom)
        b = pl.program_id(0)
        img_ref[pl.ds(b, 1), :] = img[None, None]


@jax.jit
def kernel(feature_batch, memory_bank):
    B, H, W, C_ = feature_batch.shape
    fv = feature_batch.reshape(B * H * W, C_)
    bank_t = jnp.pad(memory_bank, ((0, N_PAD - N_BANK), (0, 0))).T

    pix, img = pl.pallas_call(
        _scorer_body,
        grid=(B_IMGS, T_STEPS),
        in_specs=[
            pl.BlockSpec((HW, C), lambda b, t: (b, 0)),
            pl.BlockSpec((C, TB), lambda b, t: (0, t)),
        ],
        out_specs=[
            pl.BlockSpec((HW, 1), lambda b, t: (b, 0)),
            pl.BlockSpec((B_IMGS, 1), lambda b, t: (0, 0)),
        ],
        out_shape=[
            jax.ShapeDtypeStruct((B_IMGS * HW, 1), jnp.float32),
            jax.ShapeDtypeStruct((B_IMGS, 1), jnp.float32),
        ],
        scratch_shapes=[pltpu.VMEM((HW, TB), jnp.float32)],
        compiler_params=pltpu.CompilerParams(
            dimension_semantics=("arbitrary", "arbitrary"),
        ),
    )(fv, bank_t)

    pixel_scores = pix.reshape(B, 1, H, W)
    image_scores = img.reshape(B)
    return (pixel_scores, image_scores)


# speedup vs baseline: 9.0682x; 1.1268x over previous
import functools

import jax
import jax.numpy as jnp
from jax.experimental import pallas as pl
from jax.experimental.pallas import tpu as pltpu

B_IMGS = 2
HW = 1024
C = 128
N_BANK = 50000
G = 16
LANES = 128
TB = G * LANES
T_STEPS = (N_BANK + TB - 1) // TB
N_PAD = T_STEPS * TB
K = 9
BIG = 3.0e38


def _oems_pairs(n):
    pairs = []

    def merge(lo, n2, r):
        step = r * 2
        if step < n2:
            merge(lo, n2, step)
            merge(lo + r, n2, step)
            for i in range(lo + r, lo + n2 - r, step):
                pairs.append((i, i + r))
        else:
            pairs.append((lo, lo + r))

    def sort_range(lo, hi):
        if (hi - lo) >= 1:
            mid = lo + ((hi - lo) // 2)
            sort_range(lo, mid)
            sort_range(mid + 1, hi)
            merge(lo, hi - lo + 1, 1)

    sort_range(0, n - 1)
    return pairs


_SORT_PAIRS = _oems_pairs(G)


def _pruned_clean_ops(n, keep):
    stages = []
    d = n // 2
    while d >= 1:
        stages.append([(i, i + d)
                       for base in range(0, n, 2 * d)
                       for i in range(base, base + d)])
        d //= 2
    needed = set(range(keep))
    pruned = []
    for ops in reversed(stages):
        sp = []
        new_needed = set()
        for (i, j) in ops:
            lo, hi = i in needed, j in needed
            if lo or hi:
                sp.append((i, j, lo, hi))
                new_needed.add(i)
                new_needed.add(j)
        needed = new_needed
        pruned.append(sp)
    return list(reversed(pruned))


_CLEAN_OPS = _pruned_clean_ops(G, K)
KL = K * LANES


def _scorer_body(fv_ref, bankt_ref, pix_ref, img_ref, run_ref):
    t = pl.program_id(1)
    fv = fv_ref[...]
    bankt = bankt_ref[...]

    m2 = jnp.sum(bankt * bankt, axis=0, keepdims=True)
    col = t * TB + jax.lax.broadcasted_iota(jnp.int32, (1, TB), 1)
    m2 = jnp.where(col < N_BANK, m2, BIG)

    qm = jnp.dot(fv * jnp.float32(-2.0), bankt,
                 preferred_element_type=jnp.float32)
    d = qm + m2

    v = [d[:, j * LANES:(j + 1) * LANES] for j in range(G)]
    for (i, j) in _SORT_PAIRS:
        lo = jnp.minimum(v[i], v[j])
        hi = jnp.maximum(v[i], v[j])
        v[i] = lo
        v[j] = hi

    @pl.when(t == 0)
    def _init():
        run_ref[...] = jnp.concatenate(v[:K], axis=1)

    @pl.when(t > 0)
    def _merge():
        r = [run_ref[:, j * LANES:(j + 1) * LANES] for j in range(K)]
        c = ([jnp.minimum(r[j], v[G - 1 - j]) for j in range(K)]
             + [v[G - 1 - j] for j in range(K, G)])
        for stage in _CLEAN_OPS:
            for (i2, j2, lo_need, hi_need) in stage:
                lo = jnp.minimum(c[i2], c[j2]) if lo_need else None
                hi = jnp.maximum(c[i2], c[j2]) if hi_need else None
                if lo_need:
                    c[i2] = lo
                if hi_need:
                    c[j2] = hi
        run_ref[...] = jnp.concatenate(c[:K], axis=1)

    @pl.when(t == T_STEPS - 1)
    def _final():
        x = run_ref[...]
        q2 = jnp.sum(fv * fv, axis=1, keepdims=True)
        iota_l = jax.lax.broadcasted_iota(jnp.int32, (HW, KL), 1)
        big_i = jnp.int32(2 ** 30)

        vals = []
        for _ in range(K):
            m = jnp.min(x, axis=1, keepdims=True)
            pos = jnp.min(jnp.where(x == m, iota_l, big_i), axis=1, keepdims=True)
            x = jnp.where(iota_l == pos, BIG, x)
            vals.append(m)

        s = [jnp.sqrt(jnp.maximum(vv + q2, jnp.float32(0.0))) for vv in vals]

        pix_ref[...] = s[0]

        mx = jnp.max(s[0])
        iota_r = jax.lax.broadcasted_iota(jnp.int32, (HW, 1), 0)
        pos_r = jnp.min(jnp.where(s[0] == mx, iota_r, big_i))
        sel = [jnp.sum(jnp.where(iota_r == pos_r, si, jnp.float32(0.0)))
               for si in s]
        e = [jnp.exp(si - sel[K - 1]) for si in sel]
        denom = e[0]
        for ei in e[1:]:
            denom = denom + ei
        img = sel[0] * (jnp.float32(1.0) - e[0] / denom)
        b = pl.program_id(0)
        img_ref[pl.ds(b, 1), :] = img[None, None]


@jax.jit
def kernel(feature_batch, memory_bank):
    B, H, W, C_ = feature_batch.shape
    fv = feature_batch.reshape(B * H * W, C_)
    bank_t = jnp.pad(memory_bank, ((0, N_PAD - N_BANK), (0, 0))).T

    pix, img = pl.pallas_call(
        _scorer_body,
        grid=(B_IMGS, T_STEPS),
        in_specs=[
            pl.BlockSpec((HW, C), lambda b, t: (b, 0)),
            pl.BlockSpec((C, TB), lambda b, t: (0, t)),
        ],
        out_specs=[
            pl.BlockSpec((HW, 1), lambda b, t: (b, 0)),
            pl.BlockSpec((B_IMGS, 1), lambda b, t: (0, 0)),
        ],
        out_shape=[
            jax.ShapeDtypeStruct((B_IMGS * HW, 1), jnp.float32),
            jax.ShapeDtypeStruct((B_IMGS, 1), jnp.float32),
        ],
        scratch_shapes=[pltpu.VMEM((HW, KL), jnp.float32)],
        compiler_params=pltpu.CompilerParams(
            dimension_semantics=("arbitrary", "arbitrary"),
        ),
    )(fv, bank_t)

    pixel_scores = pix.reshape(B, 1, H, W)
    image_scores = img.reshape(B)
    return (pixel_scores, image_scores)
